# baseline (speedup 1.0000x reference)
import jax
import jax.numpy as jnp
from jax import lax
from jax.experimental import pallas as pl
from jax.experimental.pallas import tpu as pltpu

_HIDDEN = 4096
_NE = 64
_NG = 8
_GS = _NE // _NG
_TKG = 4
_SCALE = 2.5
_NT = 16384

_BT = 256


def _router_body(hs_ref, w_ref, b_ref, idx_ref, wt_ref):
    hs = hs_ref[...]
    w = w_ref[...]
    logits = lax.dot_general(hs, w, (((1,), (1,)), ((), ())),
                             preferred_element_type=jnp.float32)
    scores = jax.nn.sigmoid(logits)
    s4c = scores + b_ref[...]

    grp = s4c.reshape(_BT, _NG, _GS)
    m1 = jnp.max(grp, axis=2)
    i8 = lax.broadcasted_iota(jnp.int32, (_BT, _NG, _GS), 2)
    ismax = grp == m1[:, :, None]
    firstpos = jnp.min(jnp.where(ismax, i8, _GS), axis=2)
    m2 = jnp.max(jnp.where(i8 == firstpos[:, :, None], -jnp.inf, grp), axis=2)
    gsc = m1 + m2

    ij = lax.broadcasted_iota(jnp.int32, (_BT, _NG, _NG), 1)
    ik = lax.broadcasted_iota(jnp.int32, (_BT, _NG, _NG), 2)
    gk = gsc[:, None, :]
    gj = gsc[:, :, None]
    gcnt = (gk > gj) | ((gk == gj) & (ik < ij))
    grank = jnp.sum(gcnt.astype(jnp.int32), axis=2)
    gmaskf = (grank < _TKG).astype(jnp.float32)
    smaskf = jnp.broadcast_to(gmaskf[:, :, None], (_BT, _NG, _GS)).reshape(_BT, _NE)
    sp = jnp.where(smaskf > 0, s4c, 0.0)

    jj = lax.broadcasted_iota(jnp.int32, (_BT, _NE, _NE), 1)
    kk = lax.broadcasted_iota(jnp.int32, (_BT, _NE, _NE), 2)
    sk = sp[:, None, :]
    sj = sp[:, :, None]
    cnt = (sk > sj) | ((sk == sj) & (kk < jj))
    r = jnp.sum(cnt.astype(jnp.int32), axis=2)

    oh = r[:, :, None] == kk
    idx_out = jnp.sum(jnp.where(oh, jj, 0), axis=1)
    wsel = jnp.sum(jnp.where(oh, scores[:, :, None], 0.0), axis=1)

    denom = jnp.sum(wsel, axis=1, keepdims=True) + 1e-20
    wt = (wsel / denom) * _SCALE

    idx_ref[...] = idx_out.astype(jnp.int32)
    wt_ref[...] = wt


def kernel(hidden_states, weight, e_score_correction_bias):
    hs = hidden_states.reshape(-1, _HIDDEN).astype(jnp.float32)
    w = weight.astype(jnp.float32)
    b = e_score_correction_bias.astype(jnp.float32).reshape(1, _NE)
    grid = (_NT // _BT,)
    idx, wt = pl.pallas_call(
        _router_body,
        grid=grid,
        in_specs=[
            pl.BlockSpec((_BT, _HIDDEN), lambda i: (i, 0)),
            pl.BlockSpec((_NE, _HIDDEN), lambda i: (0, 0)),
            pl.BlockSpec((1, _NE), lambda i: (0, 0)),
        ],
        out_specs=[
            pl.BlockSpec((_BT, _NE), lambda i: (i, 0)),
            pl.BlockSpec((_BT, _NE), lambda i: (i, 0)),
        ],
        out_shape=[
            jax.ShapeDtypeStruct((_NT, _NE), jnp.int32),
            jax.ShapeDtypeStruct((_NT, _NE), jnp.float32),
        ],
        compiler_params=pltpu.CompilerParams(
            dimension_semantics=("arbitrary",),
        ),
    )(hs, w, b)
    return (idx, wt)


# speedup vs baseline: 5.6824x; 5.6824x over previous
import numpy as np
import jax
import jax.numpy as jnp
from jax import lax
from jax.experimental import pallas as pl
from jax.experimental.pallas import tpu as pltpu

_HIDDEN = 4096
_NE = 64
_NG = 8
_GS = _NE // _NG
_TKG = 4
_SCALE = 2.5
_NT = 16384

_BT = 256

_STEPS = []
for _m in (2, 4, 8, 16, 32, 64):
    _d = _m // 2
    while _d >= 1:
        _STEPS.append((_m, _d))
        _d //= 2


def _monotone_i32(x):
    u = lax.bitcast_convert_type(x, jnp.int32)
    return u ^ (lax.shift_right_arithmetic(u, 31) & jnp.int32(0x7FFFFFFF))


def _xor_swap(x, d):
    r = x.reshape(_NE // (2 * d), 2, d, x.shape[-1])
    return jnp.concatenate((r[:, 1:2], r[:, 0:1]), axis=1).reshape(x.shape)


def _router_body(hs_ref, w_ref, b_ref, idx_ref, wt_ref):
    hs = hs_ref[...]
    w = w_ref[...]
    logits = lax.dot_general(w, hs, (((1,), (1,)), ((), ())),
                             preferred_element_type=jnp.float32)
    scores = jax.nn.sigmoid(logits)
    s4c = scores + b_ref[...]

    s3 = s4c.reshape(_NG, _GS, _BT)
    m1 = jnp.max(s3, axis=1)
    i8 = lax.broadcasted_iota(jnp.int32, (_NG, _GS, _BT), 1)
    ismax = s3 == m1[:, None, :]
    firstpos = jnp.min(jnp.where(ismax, i8, _GS), axis=1)
    m2 = jnp.max(jnp.where(i8 == firstpos[:, None, :], -jnp.inf, s3), axis=1)
    gkey = _monotone_i32(m1 + m2)

    tgl = gkey[None, :, :] - gkey[:, None, :]
    ig = lax.broadcasted_iota(jnp.int32, (_NG, _NG, 1), 0)
    ih = lax.broadcasted_iota(jnp.int32, (_NG, _NG, 1), 1)
    mlt = (ih < ig).astype(jnp.int32)
    gcnt = (tgl + mlt) > 0
    grank = jnp.sum(gcnt.astype(jnp.int32), axis=1)
    gmf = (grank < _TKG).astype(jnp.float32)
    smf = jnp.broadcast_to(gmf[:, None, :], (_NG, _GS, _BT)).reshape(_NE, _BT)
    sp = jnp.where(smf > 0, s4c, 0.0)

    key = _monotone_i32(sp)
    idx = lax.broadcasted_iota(jnp.int32, (_NE, _BT), 0)
    sc = scores
    lane0 = lax.broadcasted_iota(jnp.int32, (_NE, 1), 0)
    for m, d in _STEPS:
        kf = ((lane0 & d) == 0) == ((lane0 & m) == 0)
        kp = _xor_swap(key, d)
        ip = _xor_swap(idx, d)
        scp = _xor_swap(sc, d)
        t = key - kp
        self_first = (t > 0) | ((t == 0) & (idx < ip))
        take_self = self_first == kf
        key = jnp.where(take_self, key, kp)
        idx = jnp.where(take_self, idx, ip)
        sc = jnp.where(take_self, sc, scp)

    denom = jnp.sum(sc, axis=0, keepdims=True) + 1e-20
    wt = (sc / denom) * _SCALE

    idx_ref[...] = idx
    wt_ref[...] = wt


def kernel(hidden_states, weight, e_score_correction_bias):
    hs = hidden_states.reshape(-1, _HIDDEN).astype(jnp.float32)
    w = weight.astype(jnp.float32)
    b = e_score_correction_bias.astype(jnp.float32).reshape(_NE, 1)
    grid = (_NT // _BT,)
    idx_t, wt_t = pl.pallas_call(
        _router_body,
        grid=grid,
        in_specs=[
            pl.BlockSpec((_BT, _HIDDEN), lambda i: (i, 0)),
            pl.BlockSpec((_NE, _HIDDEN), lambda i: (0, 0)),
            pl.BlockSpec((_NE, 1), lambda i: (0, 0)),
        ],
        out_specs=[
            pl.BlockSpec((_NE, _BT), lambda i: (0, i)),
            pl.BlockSpec((_NE, _BT), lambda i: (0, i)),
        ],
        out_shape=[
            jax.ShapeDtypeStruct((_NE, _NT), jnp.int32),
            jax.ShapeDtypeStruct((_NE, _NT), jnp.float32),
        ],
        compiler_params=pltpu.CompilerParams(
            dimension_semantics=("arbitrary",),
        ),
    )(hs, w, b)
    return (idx_t.T, wt_t.T)


# speedup vs baseline: 6.8835x; 1.2114x over previous
import numpy as np
import jax
import jax.numpy as jnp
from jax import lax
from jax.experimental import pallas as pl
from jax.experimental.pallas import tpu as pltpu

_HIDDEN = 4096
_NE = 64
_NG = 8
_GS = _NE // _NG
_TKG = 4
_SCALE = 2.5
_NT = 16384

_BT = 256

_STEPS = []
for _m in (2, 4, 8, 16, 32, 64):
    _d = _m // 2
    while _d >= 1:
        _STEPS.append((_m, _d))
        _d //= 2


def _monotone_i32(x):
    u = lax.bitcast_convert_type(x, jnp.int32)
    return u ^ (lax.shift_right_arithmetic(u, 31) & jnp.int32(0x7FFFFFFF))


def _swap_ax0(x, d):
    r = x.reshape(8 // (2 * d), 2, d, 8, x.shape[-1])
    return jnp.concatenate((r[:, 1:2], r[:, 0:1]), axis=1).reshape(x.shape)


def _swap_ax1(x, d):
    r = x.reshape(8, 8 // (2 * d), 2, d, x.shape[-1])
    return jnp.concatenate((r[:, :, 1:2], r[:, :, 0:1]), axis=2).reshape(x.shape)


def _router_body(hs_ref, w_ref, b_ref, idx_ref, wt_ref):
    hs = hs_ref[...]
    w = w_ref[...]
    logits = lax.dot_general(w, hs, (((1,), (1,)), ((), ())),
                             preferred_element_type=jnp.float32)
    scores = jax.nn.sigmoid(logits)
    s4c = scores + b_ref[...]

    s3 = s4c.reshape(_NG, _GS, _BT)
    m1 = jnp.max(s3, axis=1)
    i8 = lax.broadcasted_iota(jnp.int32, (_NG, _GS, _BT), 1)
    ismax = s3 == m1[:, None, :]
    firstpos = jnp.min(jnp.where(ismax, i8, _GS), axis=1)
    m2 = jnp.max(jnp.where(i8 == firstpos[:, None, :], -jnp.inf, s3), axis=1)
    gkey = _monotone_i32(m1 + m2)

    tgl = gkey[None, :, :] - gkey[:, None, :]
    ig = lax.broadcasted_iota(jnp.int32, (_NG, _NG, 1), 0)
    ih = lax.broadcasted_iota(jnp.int32, (_NG, _NG, 1), 1)
    mlt = (ih < ig).astype(jnp.int32)
    gcnt = (tgl + mlt) > 0
    grank = jnp.sum(gcnt.astype(jnp.int32), axis=1)
    gmf = (grank < _TKG).astype(jnp.float32)
    smf = jnp.broadcast_to(gmf[:, None, :], (_NG, _GS, _BT)).reshape(_NE, _BT)
    sp = jnp.where(smf > 0, s4c, 0.0)

    key = jnp.swapaxes(_monotone_i32(sp).reshape(_NG, _GS, _BT), 0, 1)
    sc = jnp.swapaxes(scores.reshape(_NG, _GS, _BT), 0, 1)
    i0 = lax.broadcasted_iota(jnp.int32, (8, 8, _BT), 0)
    i1 = lax.broadcasted_iota(jnp.int32, (8, 8, _BT), 1)
    idx = 8 * i1 + i0
    ii0 = lax.broadcasted_iota(jnp.int32, (8, 8, 1), 0)
    ii1 = lax.broadcasted_iota(jnp.int32, (8, 8, 1), 1)
    inet = 8 * ii1 + ii0
    for m, d in _STEPS:
        kf = ((inet & d) == 0) == ((inet & m) == 0)
        dirsign = jnp.where(kf, jnp.int32(1), jnp.int32(-1))
        swap = _swap_ax0 if d < 8 else _swap_ax1
        dd = d if d < 8 else d // 8
        kp = swap(key, dd)
        ip = swap(idx, dd)
        scp = swap(sc, dd)
        tl = jnp.where(idx < ip, jnp.int32(1), jnp.int32(-1))
        t_adj = (key - kp) * 2 + tl
        take_self = (t_adj * dirsign) > 0
        key = jnp.where(take_self, key, kp)
        idx = jnp.where(take_self, idx, ip)
        sc = jnp.where(take_self, sc, scp)
    idx_s = jnp.swapaxes(idx, 0, 1).reshape(_NE, _BT)
    sc_s = jnp.swapaxes(sc, 0, 1).reshape(_NE, _BT)

    denom = jnp.sum(sc_s, axis=0, keepdims=True) + 1e-20
    wt = (sc_s / denom) * _SCALE

    idx_ref[...] = idx_s
    wt_ref[...] = wt


def kernel(hidden_states, weight, e_score_correction_bias):
    hs = hidden_states.reshape(-1, _HIDDEN).astype(jnp.float32)
    w = weight.astype(jnp.float32)
    b = e_score_correction_bias.astype(jnp.float32).reshape(_NE, 1)
    grid = (_NT // _BT,)
    idx_t, wt_t = pl.pallas_call(
        _router_body,
        grid=grid,
        in_specs=[
            pl.BlockSpec((_BT, _HIDDEN), lambda i: (i, 0)),
            pl.BlockSpec((_NE, _HIDDEN), lambda i: (0, 0)),
            pl.BlockSpec((_NE, 1), lambda i: (0, 0)),
        ],
        out_specs=[
            pl.BlockSpec((_NE, _BT), lambda i: (0, i)),
            pl.BlockSpec((_NE, _BT), lambda i: (0, i)),
        ],
        out_shape=[
            jax.ShapeDtypeStruct((_NE, _NT), jnp.int32),
            jax.ShapeDtypeStruct((_NE, _NT), jnp.float32),
        ],
        compiler_params=pltpu.CompilerParams(
            dimension_semantics=("arbitrary",),
        ),
    )(hs, w, b)
    return (idx_t.T, wt_t.T)


# speedup vs baseline: 7.6470x; 1.1109x over previous
import numpy as np
import jax
import jax.numpy as jnp
from jax import lax
from jax.experimental import pallas as pl
from jax.experimental.pallas import tpu as pltpu

_HIDDEN = 4096
_NE = 64
_NG = 8
_GS = _NE // _NG
_TKG = 4
_SCALE = 2.5
_NT = 16384

_BT = 512

_STEPS = []
for _m in (2, 4, 8, 16, 32, 64):
    _d = _m // 2
    while _d >= 1:
        _STEPS.append((_m, _d))
        _d //= 2


def _monotone_i32(x):
    u = lax.bitcast_convert_type(x, jnp.int32)
    return u ^ (lax.shift_right_arithmetic(u, 31) & jnp.int32(0x7FFFFFFF))


def _swap_ax0(x, d):
    r = x.reshape(8 // (2 * d), 2, d, 8, x.shape[-1])
    return jnp.concatenate((r[:, 1:2], r[:, 0:1]), axis=1).reshape(x.shape)


def _swap_ax1(x, d):
    r = x.reshape(8, 8 // (2 * d), 2, d, x.shape[-1])
    return jnp.concatenate((r[:, :, 1:2], r[:, :, 0:1]), axis=2).reshape(x.shape)


def _router_body(hs_ref, w_ref, b_ref, idx_ref, wt_ref):
    hs = hs_ref[...]
    w = w_ref[...]
    logits = lax.dot_general(w, hs, (((1,), (1,)), ((), ())),
                             preferred_element_type=jnp.float32)
    scores = jax.nn.sigmoid(logits)
    s4c = scores + b_ref[...]

    s3 = s4c.reshape(_NG, _GS, _BT)
    m1 = jnp.max(s3, axis=1)
    i8 = lax.broadcasted_iota(jnp.int32, (_NG, _GS, _BT), 1)
    ismax = s3 == m1[:, None, :]
    firstpos = jnp.min(jnp.where(ismax, i8, _GS), axis=1)
    m2 = jnp.max(jnp.where(i8 == firstpos[:, None, :], -jnp.inf, s3), axis=1)
    gkey = _monotone_i32(m1 + m2)

    tgl = gkey[None, :, :] - gkey[:, None, :]
    ig = lax.broadcasted_iota(jnp.int32, (_NG, _NG, 1), 0)
    ih = lax.broadcasted_iota(jnp.int32, (_NG, _NG, 1), 1)
    mlt = (ih < ig).astype(jnp.int32)
    gcnt = (tgl + mlt) > 0
    grank = jnp.sum(gcnt.astype(jnp.int32), axis=1)
    gmf = (grank < _TKG).astype(jnp.float32)
    smf = jnp.broadcast_to(gmf[:, None, :], (_NG, _GS, _BT)).reshape(_NE, _BT)
    sp = jnp.where(smf > 0, s4c, 0.0)

    key = jnp.swapaxes(_monotone_i32(sp).reshape(_NG, _GS, _BT), 0, 1)
    sc = jnp.swapaxes(scores.reshape(_NG, _GS, _BT), 0, 1)
    i0 = lax.broadcasted_iota(jnp.int32, (8, 8, _BT), 0)
    i1 = lax.broadcasted_iota(jnp.int32, (8, 8, _BT), 1)
    idx = 8 * i1 + i0
    ii0 = lax.broadcasted_iota(jnp.int32, (8, 8, 1), 0)
    ii1 = lax.broadcasted_iota(jnp.int32, (8, 8, 1), 1)
    inet = 8 * ii1 + ii0
    for m, d in _STEPS:
        kf = ((inet & d) == 0) == ((inet & m) == 0)
        dirsign = jnp.where(kf, jnp.int32(1), jnp.int32(-1))
        swap = _swap_ax0 if d < 8 else _swap_ax1
        dd = d if d < 8 else d // 8
        kp = swap(key, dd)
        ip = swap(idx, dd)
        scp = swap(sc, dd)
        tl = jnp.where(idx < ip, jnp.int32(1), jnp.int32(-1))
        t_adj = (key - kp) * 2 + tl
        take_self = (t_adj * dirsign) > 0
        key = jnp.where(take_self, key, kp)
        idx = jnp.where(take_self, idx, ip)
        sc = jnp.where(take_self, sc, scp)
    idx_s = jnp.swapaxes(idx, 0, 1).reshape(_NE, _BT)
    sc_s = jnp.swapaxes(sc, 0, 1).reshape(_NE, _BT)

    denom = jnp.sum(sc_s, axis=0, keepdims=True) + 1e-20
    wt = (sc_s / denom) * _SCALE

    idx_ref[...] = idx_s
    wt_ref[...] = wt


def kernel(hidden_states, weight, e_score_correction_bias):
    hs = hidden_states.reshape(-1, _HIDDEN).astype(jnp.float32)
    w = weight.astype(jnp.float32)
    b = e_score_correction_bias.astype(jnp.float32).reshape(_NE, 1)
    grid = (_NT // _BT,)
    idx_t, wt_t = pl.pallas_call(
        _router_body,
        grid=grid,
        in_specs=[
            pl.BlockSpec((_BT, _HIDDEN), lambda i: (i, 0)),
            pl.BlockSpec((_NE, _HIDDEN), lambda i: (0, 0)),
            pl.BlockSpec((_NE, 1), lambda i: (0, 0)),
        ],
        out_specs=[
            pl.BlockSpec((_NE, _BT), lambda i: (0, i)),
            pl.BlockSpec((_NE, _BT), lambda i: (0, i)),
        ],
        out_shape=[
            jax.ShapeDtypeStruct((_NE, _NT), jnp.int32),
            jax.ShapeDtypeStruct((_NE, _NT), jnp.float32),
        ],
        compiler_params=pltpu.CompilerParams(
            dimension_semantics=("parallel",),
        ),
    )(hs, w, b)
    return (idx_t.T, wt_t.T)


# speedup vs baseline: 7.9607x; 1.0410x over previous
import numpy as np
import jax
import jax.numpy as jnp
from jax import lax
from jax.experimental import pallas as pl
from jax.experimental.pallas import tpu as pltpu

_HIDDEN = 4096
_NE = 64
_NG = 8
_GS = _NE // _NG
_TKG = 4
_SCALE = 2.5
_NT = 16384

_BT = 1024

_STEPS = []
for _m in (2, 4, 8, 16, 32, 64):
    _d = _m // 2
    while _d >= 1:
        _STEPS.append((_m, _d))
        _d //= 2


def _monotone_i32(x):
    u = lax.bitcast_convert_type(x, jnp.int32)
    return u ^ (lax.shift_right_arithmetic(u, 31) & jnp.int32(0x7FFFFFFF))


def _swap_ax0(x, d):
    r = x.reshape(8 // (2 * d), 2, d, 8, x.shape[-1])
    return jnp.concatenate((r[:, 1:2], r[:, 0:1]), axis=1).reshape(x.shape)


def _swap_ax1(x, d):
    r = x.reshape(8, 8 // (2 * d), 2, d, x.shape[-1])
    return jnp.concatenate((r[:, :, 1:2], r[:, :, 0:1]), axis=2).reshape(x.shape)


def _router_body(hs_ref, w_ref, b_ref, idx_ref, wt_ref):
    hs = hs_ref[...]
    w = w_ref[...]
    logits = lax.dot_general(w, hs, (((1,), (1,)), ((), ())),
                             preferred_element_type=jnp.float32)
    scores = jax.nn.sigmoid(logits)
    s4c = scores + b_ref[...]

    s3 = s4c.reshape(_NG, _GS, _BT)
    m1 = jnp.max(s3, axis=1)
    i8 = lax.broadcasted_iota(jnp.int32, (_NG, _GS, _BT), 1)
    ismax = s3 == m1[:, None, :]
    firstpos = jnp.min(jnp.where(ismax, i8, _GS), axis=1)
    m2 = jnp.max(jnp.where(i8 == firstpos[:, None, :], -jnp.inf, s3), axis=1)
    gkey = _monotone_i32(m1 + m2)

    tgl = gkey[None, :, :] - gkey[:, None, :]
    ig = lax.broadcasted_iota(jnp.int32, (_NG, _NG, 1), 0)
    ih = lax.broadcasted_iota(jnp.int32, (_NG, _NG, 1), 1)
    mlt = (ih < ig).astype(jnp.int32)
    gcnt = (tgl + mlt) > 0
    grank = jnp.sum(gcnt.astype(jnp.int32), axis=1)
    gmf = (grank < _TKG).astype(jnp.float32)
    smf = jnp.broadcast_to(gmf[:, None, :], (_NG, _GS, _BT)).reshape(_NE, _BT)
    sp = jnp.where(smf > 0, s4c, 0.0)

    key = jnp.swapaxes(_monotone_i32(sp).reshape(_NG, _GS, _BT), 0, 1)
    sc = jnp.swapaxes(scores.reshape(_NG, _GS, _BT), 0, 1)
    i0 = lax.broadcasted_iota(jnp.int32, (8, 8, _BT), 0)
    i1 = lax.broadcasted_iota(jnp.int32, (8, 8, _BT), 1)
    idx = 8 * i1 + i0
    ii0 = lax.broadcasted_iota(jnp.int32, (8, 8, 1), 0)
    ii1 = lax.broadcasted_iota(jnp.int32, (8, 8, 1), 1)
    inet = 8 * ii1 + ii0
    for m, d in _STEPS:
        kf = ((inet & d) == 0) == ((inet & m) == 0)
        dirsign = jnp.where(kf, jnp.int32(1), jnp.int32(-1))
        swap = _swap_ax0 if d < 8 else _swap_ax1
        dd = d if d < 8 else d // 8
        kp = swap(key, dd)
        ip = swap(idx, dd)
        scp = swap(sc, dd)
        tl = jnp.where(idx < ip, jnp.int32(1), jnp.int32(-1))
        t_adj = (key - kp) * 2 + tl
        take_self = (t_adj * dirsign) > 0
        key = jnp.where(take_self, key, kp)
        idx = jnp.where(take_self, idx, ip)
        sc = jnp.where(take_self, sc, scp)
    idx_s = jnp.swapaxes(idx, 0, 1).reshape(_NE, _BT)
    sc_s = jnp.swapaxes(sc, 0, 1).reshape(_NE, _BT)

    denom = jnp.sum(sc_s, axis=0, keepdims=True) + 1e-20
    wt = (sc_s / denom) * _SCALE

    idx_ref[...] = idx_s
    wt_ref[...] = wt


def kernel(hidden_states, weight, e_score_correction_bias):
    hs = hidden_states.reshape(-1, _HIDDEN).astype(jnp.float32)
    w = weight.astype(jnp.float32)
    b = e_score_correction_bias.astype(jnp.float32).reshape(_NE, 1)
    grid = (_NT // _BT,)
    idx_t, wt_t = pl.pallas_call(
        _router_body,
        grid=grid,
        in_specs=[
            pl.BlockSpec((_BT, _HIDDEN), lambda i: (i, 0)),
            pl.BlockSpec((_NE, _HIDDEN), lambda i: (0, 0)),
            pl.BlockSpec((_NE, 1), lambda i: (0, 0)),
        ],
        out_specs=[
            pl.BlockSpec((_NE, _BT), lambda i: (0, i)),
            pl.BlockSpec((_NE, _BT), lambda i: (0, i)),
        ],
        out_shape=[
            jax.ShapeDtypeStruct((_NE, _NT), jnp.int32),
            jax.ShapeDtypeStruct((_NE, _NT), jnp.float32),
        ],
        compiler_params=pltpu.CompilerParams(
            dimension_semantics=("parallel",),
        ),
    )(hs, w, b)
    return (idx_t.T, wt_t.T)
